# Initial kernel scaffold; baseline (speedup 1.0000x reference)
import jax
import jax.numpy as jnp
from jax.experimental import pallas as pl


def kernel(x, W, b):
    raise NotImplementedError("write your pallas kernel here")


# speedup vs baseline: 4.8301x; 4.8301x over previous
import functools

import jax
import jax.numpy as jnp
from jax import lax
from jax.experimental import pallas as pl
from jax.experimental.pallas import tpu as pltpu

_TOKENS = 8192
_DMODEL = 4096
_EXPERTS = 64
_K = 8
_BLOCK = 512

_NEG_INF = float("-inf")


def _router_block(x_ref, w_ref, b_ref, gates_ref, idx_ref):
    logits = jax.lax.dot_general(
        x_ref[...], w_ref[...],
        dimension_numbers=(((1,), (1,)), ((), ())),
        preferred_element_type=jnp.float32,
    ) + b_ref[...][None, :]

    blk = logits.shape[0]
    col = lax.broadcasted_iota(jnp.int32, (blk, _EXPERTS), 1)

    work = logits
    maxes = []
    amaxes = []
    for _ in range(_K):
        m = jnp.max(work, axis=1, keepdims=True)
        a = jnp.min(jnp.where(work == m, col, _EXPERTS), axis=1, keepdims=True)
        maxes.append(m)
        amaxes.append(a)
        work = jnp.where(col == a, _NEG_INF, work)

    exps = [jnp.exp(m - maxes[0]) for m in maxes]
    denom = functools.reduce(lambda p, q: p + q, exps)

    gates = jnp.zeros((blk, _EXPERTS), jnp.float32)
    for k in range(_K):
        gates = jnp.where(col == amaxes[k], exps[k] / denom, gates)
    gates_ref[...] = gates

    colk = lax.broadcasted_iota(jnp.int32, (blk, _K), 1)
    idx = jnp.zeros((blk, _K), jnp.int32)
    for k in range(_K):
        idx = jnp.where(colk == k, amaxes[k], idx)
    idx_ref[...] = idx


def kernel(x, W, b):
    grid = _TOKENS // _BLOCK
    gates, idx = pl.pallas_call(
        _router_block,
        grid=(grid,),
        in_specs=[
            pl.BlockSpec((_BLOCK, _DMODEL), lambda i: (i, 0)),
            pl.BlockSpec((_EXPERTS, _DMODEL), lambda i: (0, 0)),
            pl.BlockSpec((_EXPERTS,), lambda i: (0,)),
        ],
        out_specs=[
            pl.BlockSpec((_BLOCK, _EXPERTS), lambda i: (i, 0)),
            pl.BlockSpec((_BLOCK, _K), lambda i: (i, 0)),
        ],
        out_shape=[
            jax.ShapeDtypeStruct((_TOKENS, _EXPERTS), jnp.float32),
            jax.ShapeDtypeStruct((_TOKENS, _K), jnp.int32),
        ],
        compiler_params=pltpu.CompilerParams(
            dimension_semantics=("arbitrary",),
        ),
    )(x, W, b)
    return (gates, idx)
